# baseline (speedup 1.0000x reference)
import functools

import jax
import jax.numpy as jnp
from jax.experimental import pallas as pl

_TB = 2304
_NH = 8


def _stage(r, cb_hi, cb_mid, cb_lo, cb2, k):
    bf16 = jnp.bfloat16
    s = jax.lax.dot_general(r.astype(bf16), cb_hi,
                            (((1,), (1,)), ((), ())),
                            preferred_element_type=jnp.float32)
    d = (jnp.sum(r * r, axis=1, keepdims=True) - 2.0 * s) + cb2
    idx = jnp.argmin(d, axis=1).astype(jnp.int32)
    oh = (jax.lax.broadcasted_iota(jnp.int32, (r.shape[0], k), 1)
          == idx[:, None]).astype(bf16)
    q = (jnp.dot(oh, cb_hi, preferred_element_type=jnp.float32)
         + jnp.dot(oh, cb_mid, preferred_element_type=jnp.float32)
         + jnp.dot(oh, cb_lo, preferred_element_type=jnp.float32))
    return r - q, idx


def _body(nq, k, x_ref, ew_ref, eb_ref, cb_ref, dw_ref, db_ref,
          out_ref, idx_ref, closs_ref):
    bf16 = jnp.bfloat16
    x = x_ref[...]
    h = jax.nn.gelu(
        jnp.dot(x.astype(bf16), ew_ref[...].astype(bf16),
                preferred_element_type=jnp.float32)
        + eb_ref[...])
    hh = _TB // _NH
    rs = [h[j * hh:(j + 1) * hh] for j in range(_NH)]
    csum = jnp.float32(0.0)
    idxs = [[] for _ in range(_NH)]
    for i in range(nq):
        cb = cb_ref[i]
        cb2 = jnp.sum(cb * cb, axis=1)[None, :]
        cb_hi = cb.astype(bf16)
        res1 = cb - cb_hi.astype(jnp.float32)
        cb_mid = res1.astype(bf16)
        cb_lo = (res1 - cb_mid.astype(jnp.float32)).astype(bf16)
        for j in range(_NH):
            rs[j], idx = _stage(rs[j], cb_hi, cb_mid, cb_lo, cb2, k)
            idxs[j].append(idx)
        for j in range(_NH):
            csum = csum + jnp.sum(rs[j] * rs[j])
    r = jnp.concatenate(rs, axis=0)
    out_ref[...] = (jnp.dot((h - r).astype(bf16), dw_ref[...].astype(bf16),
                            preferred_element_type=jnp.float32)
                    + db_ref[...])
    idx_ref[...] = jnp.concatenate(
        [jnp.stack(ix, axis=1) for ix in idxs], axis=0)
    acc = jnp.full((8, 128), csum, jnp.float32)

    @pl.when(pl.program_id(0) == 0)
    def _init():
        closs_ref[...] = acc

    @pl.when(pl.program_id(0) != 0)
    def _accum():
        closs_ref[...] += acc


def kernel(data_object, enc_W, enc_b, codebooks, dec_W, dec_b):
    b, t, c = data_object.shape
    nq, k, d = codebooks.shape
    n = b * t
    grid = n // _TB
    x = data_object.reshape(n, c)

    out, idx, closs_acc = pl.pallas_call(
        functools.partial(_body, nq, k),
        grid=(grid,),
        in_specs=[
            pl.BlockSpec((_TB, c), lambda i: (i, 0)),
            pl.BlockSpec((c, d), lambda i: (0, 0)),
            pl.BlockSpec((1, d), lambda i: (0, 0)),
            pl.BlockSpec((nq, k, d), lambda i: (0, 0, 0)),
            pl.BlockSpec((d, c), lambda i: (0, 0)),
            pl.BlockSpec((1, c), lambda i: (0, 0)),
        ],
        out_specs=[
            pl.BlockSpec((_TB, c), lambda i: (i, 0)),
            pl.BlockSpec((_TB, nq), lambda i: (i, 0)),
            pl.BlockSpec((8, 128), lambda i: (0, 0)),
        ],
        out_shape=[
            jax.ShapeDtypeStruct((n, c), jnp.float32),
            jax.ShapeDtypeStruct((n, nq), jnp.int32),
            jax.ShapeDtypeStruct((8, 128), jnp.float32),
        ],
    )(x, enc_W, enc_b.reshape(1, d), codebooks, dec_W, dec_b.reshape(1, c))

    logits = out.reshape(b, t, c)
    closs = closs_acc[0, 0] * (1.25 / (n * d))
    return logits, closs, idx.reshape(b, t, nq)


# speedup vs baseline: 1.6139x; 1.6139x over previous
import functools

import jax
import jax.numpy as jnp
from jax.experimental import pallas as pl

_TB = 1152
_NH = 3


def _stage(r, cb_hi, cb_mid, cb_lo, cb2, k):
    bf16 = jnp.bfloat16
    s = jax.lax.dot_general(r.astype(bf16), cb_hi,
                            (((1,), (1,)), ((), ())),
                            preferred_element_type=jnp.float32)
    d = (jnp.sum(r * r, axis=1, keepdims=True) - 2.0 * s) + cb2
    idx = jnp.argmin(d, axis=1).astype(jnp.int32)
    oh = (jax.lax.broadcasted_iota(jnp.int32, (r.shape[0], k), 1)
          == idx[:, None]).astype(bf16)
    q = (jnp.dot(oh, cb_hi, preferred_element_type=jnp.float32)
         + jnp.dot(oh, cb_mid, preferred_element_type=jnp.float32)
         + jnp.dot(oh, cb_lo, preferred_element_type=jnp.float32))
    return r - q, idx


def _body(nq, k, x_ref, ew_ref, eb_ref, cb_ref, dw_ref, db_ref,
          out_ref, idx_ref, closs_ref):
    bf16 = jnp.bfloat16
    x = x_ref[...]
    h = jax.nn.gelu(
        jnp.dot(x.astype(bf16), ew_ref[...].astype(bf16),
                preferred_element_type=jnp.float32)
        + eb_ref[...])
    hh = _TB // _NH
    rs = [h[j * hh:(j + 1) * hh] for j in range(_NH)]
    csum = jnp.float32(0.0)
    idxs = [[] for _ in range(_NH)]
    for i in range(nq):
        cb = cb_ref[i]
        cb2 = jnp.sum(cb * cb, axis=1)[None, :]
        cb_hi = cb.astype(bf16)
        res1 = cb - cb_hi.astype(jnp.float32)
        cb_mid = res1.astype(bf16)
        cb_lo = (res1 - cb_mid.astype(jnp.float32)).astype(bf16)
        for j in range(_NH):
            rs[j], idx = _stage(rs[j], cb_hi, cb_mid, cb_lo, cb2, k)
            idxs[j].append(idx)
        for j in range(_NH):
            csum = csum + jnp.sum(rs[j] * rs[j])
    r = jnp.concatenate(rs, axis=0)
    out_ref[...] = (jnp.dot((h - r).astype(bf16), dw_ref[...].astype(bf16),
                            preferred_element_type=jnp.float32)
                    + db_ref[...])
    idx_ref[...] = jnp.concatenate(
        [jnp.stack(ix, axis=1) for ix in idxs], axis=0)
    acc = jnp.full((8, 128), csum, jnp.float32)

    @pl.when(pl.program_id(0) == 0)
    def _init():
        closs_ref[...] = acc

    @pl.when(pl.program_id(0) != 0)
    def _accum():
        closs_ref[...] += acc


def kernel(data_object, enc_W, enc_b, codebooks, dec_W, dec_b):
    b, t, c = data_object.shape
    nq, k, d = codebooks.shape
    n = b * t
    grid = n // _TB
    x = data_object.reshape(n, c)

    out, idx, closs_acc = pl.pallas_call(
        functools.partial(_body, nq, k),
        grid=(grid,),
        in_specs=[
            pl.BlockSpec((_TB, c), lambda i: (i, 0)),
            pl.BlockSpec((c, d), lambda i: (0, 0)),
            pl.BlockSpec((1, d), lambda i: (0, 0)),
            pl.BlockSpec((nq, k, d), lambda i: (0, 0, 0)),
            pl.BlockSpec((d, c), lambda i: (0, 0)),
            pl.BlockSpec((1, c), lambda i: (0, 0)),
        ],
        out_specs=[
            pl.BlockSpec((_TB, c), lambda i: (i, 0)),
            pl.BlockSpec((_TB, nq), lambda i: (i, 0)),
            pl.BlockSpec((8, 128), lambda i: (0, 0)),
        ],
        out_shape=[
            jax.ShapeDtypeStruct((n, c), jnp.float32),
            jax.ShapeDtypeStruct((n, nq), jnp.int32),
            jax.ShapeDtypeStruct((8, 128), jnp.float32),
        ],
    )(x, enc_W, enc_b.reshape(1, d), codebooks, dec_W, dec_b.reshape(1, c))

    logits = out.reshape(b, t, c)
    closs = closs_acc[0, 0] * (1.25 / (n * d))
    return logits, closs, idx.reshape(b, t, nq)
